# baseline (speedup 1.0000x reference)
import functools

import jax
import jax.numpy as jnp
from jax import lax
from jax.experimental import pallas as pl
from jax.experimental.pallas import tpu as pltpu
from jax.experimental.pallas import tpu_sc as plsc

N, D = 10000, 128
NC, NS = 2, 16
NW = NC * NS
CHUNK = 40
RING = 8
NSTRIPS = N // CHUNK


def _agg_body(nblocks, bch, x_hbm, src_hbm, dst_hbm, out_hbm,
              src_v, dst_v, rows_v, agg_sh, sems):
    cid = lax.axis_index("c")
    sid = lax.axis_index("s")

    def zrow(i, carry):
        for c in range(D // 16):
            rows_v[0, i, pl.ds(c * 16, 16)] = jnp.zeros((16,), jnp.float32)
        return carry
    lax.fori_loop(0, CHUNK, zrow, 0)

    def zcopy(k, carry):
        strip = sid + k * NS
        pltpu.sync_copy(rows_v.at[0],
                        agg_sh.at[pl.ds(strip * CHUNK, CHUNK)])
        return carry
    lax.fori_loop(0, (NSTRIPS - sid + NS - 1) // NS, zcopy, 0)

    plsc.subcore_barrier()

    def fire(c, p):
        pltpu.make_async_copy(
            x_hbm.at[src_v.at[c]], rows_v.at[p], sems.at[p]).start()

    def block(b, carry):
        pltpu.sync_copy(src_hbm.at[cid, sid, b], src_v)
        pltpu.sync_copy(dst_hbm.at[cid, sid, b], dst_v)
        for r in range(RING - 1):
            fire(r, r)

        def chunk(j, carry2):
            p = j % RING
            pltpu.make_async_copy(
                x_hbm.at[src_v.at[j]], rows_v.at[p], sems.at[p]).wait()

            @pl.when(j + RING - 1 < bch)
            def _prefetch():
                fire(j + RING - 1, (j + RING - 1) % RING)

            pltpu.sync_copy(rows_v.at[p], agg_sh.at[dst_v.at[j]], add=True)
            return carry2
        lax.fori_loop(0, bch, chunk, 0)
        return carry
    lax.fori_loop(0, nblocks, block, 0)

    plsc.subcore_barrier()

    def wcopy(k, carry):
        strip = sid + k * NS
        pltpu.sync_copy(
            agg_sh.at[pl.ds(strip * CHUNK, CHUNK)],
            out_hbm.at[cid, pl.ds(strip * CHUNK, CHUNK)])
        return carry
    lax.fori_loop(0, (NSTRIPS - sid + NS - 1) // NS, wcopy, 0)


@functools.cache
def _make_agg(nblocks, bch):
    return pl.kernel(
        functools.partial(_agg_body, nblocks, bch),
        out_type=jax.ShapeDtypeStruct((NC, N, D), jnp.float32),
        mesh=plsc.VectorSubcoreMesh(core_axis_name="c", subcore_axis_name="s"),
        scratch_types=[
            pltpu.VMEM((bch, CHUNK), jnp.int32),
            pltpu.VMEM((bch, CHUNK), jnp.int32),
            pltpu.VMEM((RING, CHUNK, D), jnp.float32),
            pltpu.VMEM_SHARED((N, D), jnp.float32),
            pltpu.SemaphoreType.DMA((RING,)),
        ],
    )


def _mlp_body(x_ref, p0_ref, p1_ref, w1_ref, b1_ref, w2_ref, b2_ref, o_ref):
    h = x_ref[...] + p0_ref[...] + p1_ref[...]
    h = jnp.dot(h, w1_ref[...], preferred_element_type=jnp.float32) + b1_ref[...]
    h = jnp.maximum(h, 0.0)
    o_ref[...] = (jnp.dot(h, w2_ref[...], preferred_element_type=jnp.float32)
                  + b2_ref[...])


_MLP_BLK = 1000


def _mlp(x, p0, p1, W1, b1, W2, b2):
    row_spec = pl.BlockSpec((_MLP_BLK, D), lambda i: (i, 0))
    full_spec = pl.BlockSpec((D, D), lambda i: (0, 0))
    bias_spec = pl.BlockSpec((1, D), lambda i: (0, 0))
    return pl.pallas_call(
        _mlp_body,
        grid=(N // _MLP_BLK,),
        in_specs=[row_spec, row_spec, row_spec,
                  full_spec, bias_spec, full_spec, bias_spec],
        out_specs=row_spec,
        out_shape=jax.ShapeDtypeStruct((N, D), jnp.float32),
    )(x, p0, p1, W1, b1.reshape(1, D), W2, b2.reshape(1, D))


def kernel(x, edge_index, W1, b1, W2, b2):
    src = edge_index[0].reshape(NC, NS, -1, CHUNK)
    dst = edge_index[1].reshape(NC, NS, -1, CHUNK)
    nchunks = src.shape[2]
    bch = next(b for b in (25, 20, 10, 5, 1) if nchunks % b == 0)
    nblocks = nchunks // bch
    src = src.reshape(NC, NS, nblocks, bch, CHUNK)
    dst = dst.reshape(NC, NS, nblocks, bch, CHUNK)
    partials = _make_agg(nblocks, bch)(x, src, dst)
    return _mlp(x, partials[0], partials[1], W1, b1, W2, b2)


# speedup vs baseline: 1.2375x; 1.2375x over previous
import functools

import jax
import jax.numpy as jnp
from jax import lax
from jax.experimental import pallas as pl
from jax.experimental.pallas import tpu as pltpu
from jax.experimental.pallas import tpu_sc as plsc

N, D = 10000, 128
NC, NS = 2, 16
NW = NC * NS
CHUNK = 80
RING = 4
NSTRIPS = N // CHUNK


def _agg_body(nblocks, bch, x_hbm, src_hbm, dst_hbm, out_hbm,
              src_v, dst_v, rows_v, agg_sh, sems):
    cid = lax.axis_index("c")
    sid = lax.axis_index("s")

    def fire(c, p):
        pltpu.make_async_copy(
            x_hbm.at[src_v.at[c]], rows_v.at[p], sems.at[p]).start()

    pltpu.sync_copy(src_hbm.at[cid, sid, 0], src_v)
    pltpu.sync_copy(dst_hbm.at[cid, sid, 0], dst_v)
    for r in range(RING - 1):
        fire(r, r)

    def zrow(i, carry):
        for c in range(D // 16):
            rows_v[RING - 1, i, pl.ds(c * 16, 16)] = (
                jnp.zeros((16,), jnp.float32))
        return carry
    lax.fori_loop(0, CHUNK, zrow, 0)

    def zcopy(k, carry):
        strip = sid + k * NS
        pltpu.sync_copy(rows_v.at[RING - 1],
                        agg_sh.at[pl.ds(strip * CHUNK, CHUNK)])
        return carry
    lax.fori_loop(0, (NSTRIPS - sid + NS - 1) // NS, zcopy, 0)

    plsc.subcore_barrier()

    def block(b, carry):
        @pl.when(b > 0)
        def _stage():
            pltpu.sync_copy(src_hbm.at[cid, sid, b], src_v)
            pltpu.sync_copy(dst_hbm.at[cid, sid, b], dst_v)
            for r in range(RING - 1):
                fire(r, r)

        def chunk(j, carry2):
            p = j % RING
            pltpu.make_async_copy(
                x_hbm.at[src_v.at[j]], rows_v.at[p], sems.at[p]).wait()

            @pl.when(j + RING - 1 < bch)
            def _prefetch():
                fire(j + RING - 1, (j + RING - 1) % RING)

            pltpu.sync_copy(rows_v.at[p], agg_sh.at[dst_v.at[j]], add=True)
            return carry2
        lax.fori_loop(0, bch, chunk, 0)
        return carry
    lax.fori_loop(0, nblocks, block, 0)

    plsc.subcore_barrier()

    def wcopy(k, carry):
        strip = sid + k * NS
        pltpu.sync_copy(
            agg_sh.at[pl.ds(strip * CHUNK, CHUNK)],
            out_hbm.at[cid, pl.ds(strip * CHUNK, CHUNK)])
        return carry
    lax.fori_loop(0, (NSTRIPS - sid + NS - 1) // NS, wcopy, 0)


@functools.cache
def _make_agg(nblocks, bch):
    return pl.kernel(
        functools.partial(_agg_body, nblocks, bch),
        out_type=jax.ShapeDtypeStruct((NC, N, D), jnp.float32),
        mesh=plsc.VectorSubcoreMesh(core_axis_name="c", subcore_axis_name="s"),
        scratch_types=[
            pltpu.VMEM((bch, CHUNK), jnp.int32),
            pltpu.VMEM((bch, CHUNK), jnp.int32),
            pltpu.VMEM((RING, CHUNK, D), jnp.float32),
            pltpu.VMEM_SHARED((N, D), jnp.float32),
            pltpu.SemaphoreType.DMA((RING,)),
        ],
    )


def _mlp_body(x_ref, p0_ref, p1_ref, w1_ref, b1_ref, w2_ref, b2_ref, o_ref):
    h = x_ref[...] + p0_ref[0] + p1_ref[0]
    h = jnp.dot(h, w1_ref[...], preferred_element_type=jnp.float32) + b1_ref[...]
    h = jnp.maximum(h, 0.0)
    o_ref[...] = (jnp.dot(h, w2_ref[...], preferred_element_type=jnp.float32)
                  + b2_ref[...])


_MLP_BLK = 1000


def _mlp(x, partials, W1, b1, W2, b2):
    row_spec = pl.BlockSpec((_MLP_BLK, D), lambda i: (i, 0))
    p0_spec = pl.BlockSpec((1, _MLP_BLK, D), lambda i: (0, i, 0))
    p1_spec = pl.BlockSpec((1, _MLP_BLK, D), lambda i: (1, i, 0))
    full_spec = pl.BlockSpec((D, D), lambda i: (0, 0))
    bias_spec = pl.BlockSpec((1, D), lambda i: (0, 0))
    return pl.pallas_call(
        _mlp_body,
        grid=(N // _MLP_BLK,),
        in_specs=[row_spec, p0_spec, p1_spec,
                  full_spec, bias_spec, full_spec, bias_spec],
        out_specs=row_spec,
        out_shape=jax.ShapeDtypeStruct((N, D), jnp.float32),
    )(x, partials, partials, W1, b1.reshape(1, D), W2, b2.reshape(1, D))


def kernel(x, edge_index, W1, b1, W2, b2):
    src = edge_index[0].reshape(NC, NS, -1, CHUNK)
    dst = edge_index[1].reshape(NC, NS, -1, CHUNK)
    nchunks = src.shape[2]
    bch = next(b for b in (25, 20, 10, 5, 1) if nchunks % b == 0)
    nblocks = nchunks // bch
    src = src.reshape(NC, NS, nblocks, bch, CHUNK)
    dst = dst.reshape(NC, NS, nblocks, bch, CHUNK)
    partials = _make_agg(nblocks, bch)(x, src, dst)
    return _mlp(x, partials, W1, b1, W2, b2)


# speedup vs baseline: 1.2742x; 1.0297x over previous
import functools

import jax
import jax.numpy as jnp
from jax import lax
from jax.experimental import pallas as pl
from jax.experimental.pallas import tpu as pltpu
from jax.experimental.pallas import tpu_sc as plsc

N, D = 10000, 128
NC, NS = 2, 16
NW = NC * NS
CHUNK = 80
RING = 3
NSTRIPS = N // CHUNK


def _agg_body(nblocks, bch, x_hbm, src_hbm, dst_hbm, out_hbm,
              src_v, dst_v, rows_v, agg_sh, sems, isems):
    cid = lax.axis_index("c")
    sid = lax.axis_index("s")
    total = nblocks * bch

    def fire(c, p):
        bf = c // bch
        pltpu.make_async_copy(
            x_hbm.at[src_v.at[bf % 2, c % bch]], rows_v.at[p],
            sems.at[p]).start()

    pltpu.sync_copy(src_hbm.at[cid, sid, 0], src_v.at[0])
    pltpu.sync_copy(dst_hbm.at[cid, sid, 0], dst_v.at[0])
    for r in range(RING - 1):
        fire(r, r)

    def zrow(i, carry):
        for c in range(D // 16):
            rows_v[RING - 1, i, pl.ds(c * 16, 16)] = (
                jnp.zeros((16,), jnp.float32))
        return carry
    lax.fori_loop(0, CHUNK, zrow, 0)

    def zcopy(k, carry):
        strip = sid + k * NS
        pltpu.sync_copy(rows_v.at[RING - 1],
                        agg_sh.at[pl.ds(strip * CHUNK, CHUNK)])
        return carry
    lax.fori_loop(0, (NSTRIPS - sid + NS - 1) // NS, zcopy, 0)

    plsc.subcore_barrier()

    def stage_start(b, q):
        pltpu.make_async_copy(
            src_hbm.at[cid, sid, b], src_v.at[q], isems.at[0]).start()
        pltpu.make_async_copy(
            dst_hbm.at[cid, sid, b], dst_v.at[q], isems.at[1]).start()

    def stage_wait(b, q):
        pltpu.make_async_copy(
            src_hbm.at[cid, sid, b], src_v.at[q], isems.at[0]).wait()
        pltpu.make_async_copy(
            dst_hbm.at[cid, sid, b], dst_v.at[q], isems.at[1]).wait()

    def chunk(j, carry):
        b = j // bch
        jj = j % bch
        q = b % 2
        p = j % RING

        @pl.when((jj == 0) & (b + 1 < nblocks))
        def _stage_next():
            stage_start(b + 1, 1 - q)

        pltpu.make_async_copy(
            x_hbm.at[src_v.at[q, jj]], rows_v.at[p], sems.at[p]).wait()

        @pl.when((jj == bch - RING) & (b + 1 < nblocks))
        def _stage_done():
            stage_wait(b + 1, 1 - q)

        @pl.when(j + RING - 1 < total)
        def _prefetch():
            fire(j + RING - 1, (j + RING - 1) % RING)

        pltpu.sync_copy(rows_v.at[p], agg_sh.at[dst_v.at[q, jj]], add=True)
        return carry
    lax.fori_loop(0, total, chunk, 0)

    plsc.subcore_barrier()

    def wcopy(k, carry):
        strip = sid + k * NS
        pltpu.sync_copy(
            agg_sh.at[pl.ds(strip * CHUNK, CHUNK)],
            out_hbm.at[cid, pl.ds(strip * CHUNK, CHUNK)])
        return carry
    lax.fori_loop(0, (NSTRIPS - sid + NS - 1) // NS, wcopy, 0)


@functools.cache
def _make_agg(nblocks, bch):
    return pl.kernel(
        functools.partial(_agg_body, nblocks, bch),
        out_type=jax.ShapeDtypeStruct((NC, N, D), jnp.float32),
        mesh=plsc.VectorSubcoreMesh(core_axis_name="c", subcore_axis_name="s"),
        scratch_types=[
            pltpu.VMEM((2, bch, CHUNK), jnp.int32),
            pltpu.VMEM((2, bch, CHUNK), jnp.int32),
            pltpu.VMEM((RING, CHUNK, D), jnp.float32),
            pltpu.VMEM_SHARED((N, D), jnp.float32),
            pltpu.SemaphoreType.DMA((RING,)),
            pltpu.SemaphoreType.DMA((2,)),
        ],
    )


def _mlp_body(x_ref, p0_ref, p1_ref, w1_ref, b1_ref, w2_ref, b2_ref, o_ref):
    h = x_ref[...] + p0_ref[0] + p1_ref[0]
    h = jnp.dot(h, w1_ref[...], preferred_element_type=jnp.float32) + b1_ref[...]
    h = jnp.maximum(h, 0.0)
    o_ref[...] = (jnp.dot(h, w2_ref[...], preferred_element_type=jnp.float32)
                  + b2_ref[...])


_MLP_BLK = 1000


def _mlp(x, partials, W1, b1, W2, b2):
    row_spec = pl.BlockSpec((_MLP_BLK, D), lambda i: (i, 0))
    p0_spec = pl.BlockSpec((1, _MLP_BLK, D), lambda i: (0, i, 0))
    p1_spec = pl.BlockSpec((1, _MLP_BLK, D), lambda i: (1, i, 0))
    full_spec = pl.BlockSpec((D, D), lambda i: (0, 0))
    bias_spec = pl.BlockSpec((1, D), lambda i: (0, 0))
    return pl.pallas_call(
        _mlp_body,
        grid=(N // _MLP_BLK,),
        in_specs=[row_spec, p0_spec, p1_spec,
                  full_spec, bias_spec, full_spec, bias_spec],
        out_specs=row_spec,
        out_shape=jax.ShapeDtypeStruct((N, D), jnp.float32),
    )(x, partials, partials, W1, b1.reshape(1, D), W2, b2.reshape(1, D))


def kernel(x, edge_index, W1, b1, W2, b2):
    src = edge_index[0].reshape(NC, NS, -1, CHUNK)
    dst = edge_index[1].reshape(NC, NS, -1, CHUNK)
    nchunks = src.shape[2]
    bch = next(b for b in (25, 20, 10, 5, 1) if nchunks % b == 0)
    if bch <= RING:
        bch = nchunks
    nblocks = nchunks // bch
    src = src.reshape(NC, NS, nblocks, bch, CHUNK)
    dst = dst.reshape(NC, NS, nblocks, bch, CHUNK)
    partials = _make_agg(nblocks, bch)(x, src, dst)
    return _mlp(x, partials, W1, b1, W2, b2)
